# baseline (speedup 1.0000x reference)
import functools

import jax
import jax.numpy as jnp
from jax import lax
from jax.experimental import pallas as pl
from jax.experimental.pallas import tpu as pltpu
from jax.experimental.pallas import tpu_sc as plsc

NWORDS = 1000000
NTAGS = 1000
EMB = 64
HID = 128
SEQ = 200

LANES = 16
SLAB = 128
NUM_W = 32
BASE_WORDS = 6
EXTRA_FROM = 24
MAX_WORDS = 7


def _gather_sum_sc(tableT, words):
    mesh = plsc.VectorSubcoreMesh(core_axis_name="c", subcore_axis_name="s")

    @functools.partial(
        pl.kernel,
        out_type=jax.ShapeDtypeStruct((NUM_W, EMB), jnp.float32),
        mesh=mesh,
        scratch_types=[
            pltpu.VMEM((LANES,), jnp.int32),
            pltpu.VMEM((MAX_WORDS, EMB, SLAB), jnp.float32),
            pltpu.VMEM((EMB,), jnp.float32),
            pltpu.SemaphoreType.DMA,
        ],
        compiler_params=pltpu.CompilerParams(needs_layout_passes=False),
    )
    def k(table_hbm, idx_hbm, out_hbm, idx_v, slabs_v, acc_v, sem):
        num_cores = lax.axis_size("c")
        wid = lax.axis_index("s") * num_cores + lax.axis_index("c")

        has7 = wid >= EXTRA_FROM
        base = BASE_WORDS * wid + jnp.maximum(wid - EXTRA_FROM, 0)
        clamped = pl.multiple_of(
            jnp.minimum((base >> 3) << 3, SEQ - LANES), 8)
        off = base - clamped
        pltpu.sync_copy(idx_hbm.at[pl.ds(clamped, LANES)], idx_v)
        lanes = []
        for kk in range(MAX_WORDS):
            w = plsc.load_gather(
                idx_v, [jnp.full((LANES,), kk + off, jnp.int32)])[0]
            start = pl.multiple_of((w >> 7) << 7, SLAB)
            if kk < BASE_WORDS:
                pltpu.async_copy(
                    table_hbm.at[:, pl.ds(start, SLAB)], slabs_v.at[kk], sem)
            else:
                @pl.when(has7)
                def _():
                    pltpu.async_copy(
                        table_hbm.at[:, pl.ds(start, SLAB)], slabs_v.at[kk],
                        sem)
            lanes.append(w - start)
        eranges = [lax.iota(jnp.int32, LANES) + d * LANES
                   for d in range(EMB // LANES)]
        accs = [jnp.zeros((LANES,), jnp.float32)
                for _ in range(EMB // LANES)]
        for kk in range(MAX_WORDS):
            if kk < BASE_WORDS:
                pltpu.make_async_copy(
                    table_hbm.at[:, pl.ds(0, SLAB)], slabs_v.at[kk],
                    sem).wait()
            else:
                @pl.when(has7)
                def _():
                    pltpu.make_async_copy(
                        table_hbm.at[:, pl.ds(0, SLAB)], slabs_v.at[kk],
                        sem).wait()
            lane = jnp.full((LANES,), lanes[kk], jnp.int32)
            kidx = jnp.full((LANES,), kk, jnp.int32)
            for d in range(EMB // LANES):
                val = plsc.load_gather(slabs_v, [kidx, eranges[d], lane])
                if kk >= BASE_WORDS:
                    val = jnp.where(has7, val, jnp.zeros_like(val))
                accs[d] = accs[d] + val
        for d in range(EMB // LANES):
            acc_v[pl.ds(d * LANES, LANES)] = accs[d]
        pltpu.sync_copy(acc_v, out_hbm.at[wid])

    return k(tableT, words)


def _mlp_body(p_ref, w0_ref, b0_ref, w1_ref, b1_ref, wo_ref, bo_ref, o_ref):
    cdims = (((1,), (1,)), ((), ()))
    h = jnp.sum(p_ref[...], axis=0, keepdims=True)
    h = jnp.tanh(
        lax.dot_general(h, w0_ref[...], cdims, preferred_element_type=jnp.float32)
        + b0_ref[...])
    h = jnp.tanh(
        lax.dot_general(h, w1_ref[...], cdims, preferred_element_type=jnp.float32)
        + b1_ref[...])
    o_ref[...] = (
        lax.dot_general(h, wo_ref[...], cdims, preferred_element_type=jnp.float32)
        + bo_ref[...])


def _mlp_tc(partials, W0, b0, W1, b1, W_out, b_out):
    return pl.pallas_call(
        _mlp_body,
        out_shape=jax.ShapeDtypeStruct((1, NTAGS), jnp.float32),
    )(partials, W0, b0, W1, b1, W_out, b_out)


def kernel(words, emb_table, W0, b0, W1, b1, W_out, b_out):
    tableT = emb_table.T
    partials = _gather_sum_sc(tableT, words.astype(jnp.int32))
    return _mlp_tc(partials, W0, b0.reshape(1, HID), W1, b1.reshape(1, HID),
                   W_out, b_out.reshape(1, NTAGS))
